# baseline (speedup 1.0000x reference)
import jax
import jax.numpy as jnp
from jax.experimental import pallas as pl
from jax.experimental.pallas import tpu as pltpu

N = 4096
D = 128
H1 = 256
H2 = 64
ODIM = 32
K = 10

BRC = 256
NB = N // BRC
PL = 2
BRL = N // PL

S_L1 = NB
S_L2 = S_L1 + PL
STEPS = S_L2 + PL


def _branch_kernel(al_ref, ar_ref, x_ref, w1_ref, w2_ref, h_ref,
                   abf, xpb, dinv, m2b):
    s = pl.program_id(0)

    @pl.when(s < S_L1)
    def _():
        rows = pl.ds(s * BRC, BRC)
        al = al_ref[...].astype(jnp.bfloat16)
        ar = ar_ref[...].astype(jnp.bfloat16)
        abf[rows, : N // 2] = al
        abf[rows, N // 2 :] = ar
        ones = jnp.ones((N // 2, 8), jnp.bfloat16)
        d = (jnp.dot(al, ones, preferred_element_type=jnp.float32)[:, :1]
             + jnp.dot(ar, ones, preferred_element_type=jnp.float32)[:, :1] + 1.0)
        dv = jax.lax.rsqrt(jnp.maximum(d, 1e-12))
        dinv[rows, :] = dv
        xpb[rows, :] = (x_ref[...] * dv).astype(jnp.bfloat16)

    @pl.when((s >= S_L1) & (s < S_L2))
    def _():
        k = s - S_L1
        rows = pl.ds(k * BRL, BRL)
        a = abf[rows, :]
        t = jnp.dot(a, xpb[...], preferred_element_type=jnp.float32)
        dv = dinv[rows, :]
        t = (t + xpb[rows, :]) * dv
        r = jnp.maximum(jnp.dot(t, w1_ref[...], preferred_element_type=jnp.float32), 0.0)
        m2 = jnp.dot(r, w2_ref[...], preferred_element_type=jnp.float32) * dv
        m2b[rows, :] = m2.astype(jnp.bfloat16)

    @pl.when(s >= S_L2)
    def _():
        k = s - S_L2
        rows = pl.ds(k * BRL, BRL)
        a = abf[rows, :]
        t = jnp.dot(a, m2b[...], preferred_element_type=jnp.float32)
        h_ref[...] = (t + m2b[rows, :]) * dinv[rows, :]


def _branch(A, X, W1, W2):
    return pl.pallas_call(
        _branch_kernel,
        grid=(STEPS,),
        in_specs=[
            pl.BlockSpec((BRC, N // 2), lambda s: (jnp.where(s < S_L1, s, S_L1 - 1), 0)),
            pl.BlockSpec((BRC, N // 2), lambda s: (jnp.where(s < S_L1, s, S_L1 - 1), 1)),
            pl.BlockSpec((BRC, D), lambda s: (jnp.where(s < S_L1, s, S_L1 - 1), 0)),
            pl.BlockSpec((D, H1), lambda s: (0, 0)),
            pl.BlockSpec((H1, H2), lambda s: (0, 0)),
        ],
        out_specs=pl.BlockSpec((BRL, H2), lambda s: (jnp.clip(s - S_L2, 0, PL - 1), 0)),
        out_shape=jax.ShapeDtypeStruct((N, H2), jnp.float32),
        scratch_shapes=[
            pltpu.VMEM((N, N), jnp.bfloat16),
            pltpu.VMEM((N, D), jnp.bfloat16),
            pltpu.VMEM((N, 1), jnp.float32),
            pltpu.VMEM((N, H2), jnp.bfloat16),
        ],
    )(A, A, X, W1, W2)


def _epilogue_kernel(hv_ref, hg_ref, wd1_ref, wd2_ref, wf_ref, bf_ref, c_ref,
                     h_ref, q_ref, p_ref, xhat_ref):
    hv = hv_ref[...]
    hg = hg_ref[...]
    r = jnp.maximum(jnp.dot(hv, wd1_ref[...], preferred_element_type=jnp.float32), 0.0)
    xhat_ref[...] = jnp.dot(r, wd2_ref[...], preferred_element_type=jnp.float32)
    wf = wf_ref[...]
    t = (jnp.dot(hv, wf[:H2], preferred_element_type=jnp.float32)
         + jnp.dot(hg, wf[H2:], preferred_element_type=jnp.float32)
         + bf_ref[...])
    h = jnp.tanh(t)
    h_ref[...] = h
    c = c_ref[...]
    cross = jnp.dot(h, c.T, preferred_element_type=jnp.float32)
    dist2 = (jnp.sum(h * h, axis=1, keepdims=True)
             + jnp.sum(c * c, axis=1)[None, :]
             - 2.0 * cross)
    q = 1.0 / (1.0 + dist2)
    q = q / jnp.sum(q, axis=1, keepdims=True)
    q_ref[...] = q
    f = jnp.sum(q, axis=0, keepdims=True)
    p = (q * q) / f
    p_ref[...] = p / jnp.sum(p, axis=1, keepdims=True)


def _epilogue(h_v, h_g, Wd1, Wd2, Wf, bf, centers):
    return pl.pallas_call(
        _epilogue_kernel,
        out_shape=[
            jax.ShapeDtypeStruct((N, ODIM), jnp.float32),
            jax.ShapeDtypeStruct((N, K), jnp.float32),
            jax.ShapeDtypeStruct((N, K), jnp.float32),
            jax.ShapeDtypeStruct((N, D), jnp.float32),
        ],
    )(h_v, h_g, Wd1, Wd2, Wf, bf, centers)


def kernel(X, adj_v, adj_glo, W1_v, W2_v, W1_g, W2_g, Wd1, Wd2, Wf, bf, centers):
    h_v = _branch(adj_v, X, W1_v, W2_v)
    h_g = _branch(adj_glo, X, W1_g, W2_g)
    h, q, p, X_hat = _epilogue(h_v, h_g, Wd1, Wd2, Wf, bf.reshape(1, ODIM), centers)
    return (h, q, p, X_hat)


# speedup vs baseline: 1.0882x; 1.0882x over previous
import jax
import jax.numpy as jnp
from jax.experimental import pallas as pl
from jax.experimental.pallas import tpu as pltpu

N = 4096
D = 128
H1 = 256
H2 = 64
ODIM = 32
K = 10

BRC = 512
NB = N // BRC
PL = 2
BRL = N // PL

S_L1 = NB
S_L2 = S_L1 + PL
STEPS = S_L2 + PL


def _branch_kernel(a_ref, x_ref, w1_ref, w2_ref, h_ref,
                   abf, xpb, dinv, m2b):
    s = pl.program_id(0)

    @pl.when(s < S_L1)
    def _():
        rows = pl.ds(s * BRC, BRC)
        a = a_ref[...].astype(jnp.bfloat16)
        abf[rows, :] = a
        ones = jnp.ones((N, 8), jnp.bfloat16)
        d = jnp.dot(a, ones, preferred_element_type=jnp.float32)[:, :1] + 1.0
        dv = jax.lax.rsqrt(jnp.maximum(d, 1e-12))
        dinv[rows, :] = dv
        xpb[rows, :] = (x_ref[...] * dv).astype(jnp.bfloat16)

    @pl.when((s >= S_L1) & (s < S_L2))
    def _():
        k = s - S_L1
        rows = pl.ds(k * BRL, BRL)
        a = abf[rows, :]
        t = jnp.dot(a, xpb[...], preferred_element_type=jnp.float32)
        dv = dinv[rows, :]
        t = (t + xpb[rows, :]) * dv
        r = jnp.maximum(jnp.dot(t, w1_ref[...], preferred_element_type=jnp.float32), 0.0)
        m2 = jnp.dot(r, w2_ref[...], preferred_element_type=jnp.float32) * dv
        m2b[rows, :] = m2.astype(jnp.bfloat16)

    @pl.when(s >= S_L2)
    def _():
        k = s - S_L2
        rows = pl.ds(k * BRL, BRL)
        a = abf[rows, :]
        t = jnp.dot(a, m2b[...], preferred_element_type=jnp.float32)
        h_ref[...] = (t + m2b[rows, :]) * dinv[rows, :]


def _branch(A, X, W1, W2):
    return pl.pallas_call(
        _branch_kernel,
        grid=(STEPS,),
        in_specs=[
            pl.BlockSpec((BRC, N), lambda s: (jnp.where(s < S_L1, s, S_L1 - 1), 0)),
            pl.BlockSpec((BRC, D), lambda s: (jnp.where(s < S_L1, s, S_L1 - 1), 0)),
            pl.BlockSpec((D, H1), lambda s: (0, 0)),
            pl.BlockSpec((H1, H2), lambda s: (0, 0)),
        ],
        out_specs=pl.BlockSpec((BRL, H2), lambda s: (jnp.clip(s - S_L2, 0, PL - 1), 0)),
        out_shape=jax.ShapeDtypeStruct((N, H2), jnp.float32),
        scratch_shapes=[
            pltpu.VMEM((N, N), jnp.bfloat16),
            pltpu.VMEM((N, D), jnp.bfloat16),
            pltpu.VMEM((N, 1), jnp.float32),
            pltpu.VMEM((N, H2), jnp.bfloat16),
        ],
        compiler_params=pltpu.CompilerParams(vmem_limit_bytes=66584576),
    )(A, X, W1, W2)


def _epilogue_kernel(hv_ref, hg_ref, wd1_ref, wd2_ref, wf_ref, bf_ref, c_ref,
                     h_ref, q_ref, p_ref, xhat_ref):
    hv = hv_ref[...]
    hg = hg_ref[...]
    r = jnp.maximum(jnp.dot(hv, wd1_ref[...], preferred_element_type=jnp.float32), 0.0)
    xhat_ref[...] = jnp.dot(r, wd2_ref[...], preferred_element_type=jnp.float32)
    wf = wf_ref[...]
    t = (jnp.dot(hv, wf[:H2], preferred_element_type=jnp.float32)
         + jnp.dot(hg, wf[H2:], preferred_element_type=jnp.float32)
         + bf_ref[...])
    h = jnp.tanh(t)
    h_ref[...] = h
    c = c_ref[...]
    cross = jnp.dot(h, c.T, preferred_element_type=jnp.float32)
    dist2 = (jnp.sum(h * h, axis=1, keepdims=True)
             + jnp.sum(c * c, axis=1)[None, :]
             - 2.0 * cross)
    q = 1.0 / (1.0 + dist2)
    q = q / jnp.sum(q, axis=1, keepdims=True)
    q_ref[...] = q
    f = jnp.sum(q, axis=0, keepdims=True)
    p = (q * q) / f
    p_ref[...] = p / jnp.sum(p, axis=1, keepdims=True)


def _epilogue(h_v, h_g, Wd1, Wd2, Wf, bf, centers):
    return pl.pallas_call(
        _epilogue_kernel,
        out_shape=[
            jax.ShapeDtypeStruct((N, ODIM), jnp.float32),
            jax.ShapeDtypeStruct((N, K), jnp.float32),
            jax.ShapeDtypeStruct((N, K), jnp.float32),
            jax.ShapeDtypeStruct((N, D), jnp.float32),
        ],
    )(h_v, h_g, Wd1, Wd2, Wf, bf, centers)


def kernel(X, adj_v, adj_glo, W1_v, W2_v, W1_g, W2_g, Wd1, Wd2, Wf, bf, centers):
    h_v = _branch(adj_v, X, W1_v, W2_v)
    h_g = _branch(adj_glo, X, W1_g, W2_g)
    h, q, p, X_hat = _epilogue(h_v, h_g, Wd1, Wd2, Wf, bf.reshape(1, ODIM), centers)
    return (h, q, p, X_hat)
